# baseline (speedup 1.0000x reference)
import functools

import jax
import jax.numpy as jnp
from jax import lax
from jax.experimental import pallas as pl
from jax.experimental.pallas import tpu as pltpu
from jax.experimental.pallas import tpu_sc as plsc

N_TOK = 16384
N_EMB = 8192
DIM = 256
TM = 256
COMMIT = 0.25

NC = 2
NS = 16
NW = NC * NS
BPW = N_TOK // NW
CH = 128


def _argmin_body(x_ref, wt_ref, idx_ref, b_ref):
    i = pl.program_id(0)

    @pl.when(i == 0)
    def _():
        wt = wt_ref[...]
        b_ref[...] = jnp.sum(wt * wt, axis=0, keepdims=True)

    x = x_ref[...]
    a = jnp.sum(x * x, axis=1, keepdims=True)
    m2 = jax.lax.dot_general(
        x + x, wt_ref[...], (((1,), (0,)), ((), ())),
        preferred_element_type=jnp.float32,
    )
    d = (a + b_ref[...]) - m2
    rowmin = jnp.min(d, axis=1, keepdims=True)
    ids = jax.lax.broadcasted_iota(jnp.int32, d.shape, 1)
    k = d.shape[1]
    idx = jnp.min(jnp.where(d == rowmin, ids, k), axis=1)
    idx_ref[...] = idx[:, None]


def _argmin_call(x, wt):
    n, dim = x.shape
    k = wt.shape[1]
    return pl.pallas_call(
        _argmin_body,
        grid=(n // TM,),
        in_specs=[
            pl.BlockSpec((TM, dim), lambda i: (i, 0)),
            pl.BlockSpec((dim, k), lambda i: (0, 0)),
        ],
        out_specs=pl.BlockSpec((TM, 1), lambda i: (i, 0)),
        out_shape=jax.ShapeDtypeStruct((n, 1), jnp.int32),
        scratch_shapes=[pltpu.VMEM((1, k), jnp.float32)],
    )(x, wt)


@functools.partial(
    pl.kernel,
    mesh=plsc.VectorSubcoreMesh(core_axis_name="c", subcore_axis_name="s"),
    out_type=[
        jax.ShapeDtypeStruct((N_TOK, DIM), jnp.float32),
        jax.ShapeDtypeStruct((NW, 16), jnp.float32),
    ],
    scratch_types=[
        pltpu.VMEM((CH,), jnp.int32),
        pltpu.VMEM((CH, DIM), jnp.float32),
        pltpu.VMEM((CH, DIM), jnp.float32),
        pltpu.VMEM((16,), jnp.float32),
        pltpu.SemaphoreType.DMA,
    ],
)
def _sc_phase2(w_hbm, x_hbm, idx_hbm, qst_hbm, loss_hbm,
               idx_v, rows_v, x_v, acc_v, sem):
    wid = lax.axis_index("s") * NC + lax.axis_index("c")
    base = wid * BPW
    acc_v[...] = jnp.zeros((16,), jnp.float32)

    def chunk(c, carry):
        cb = base + c * CH
        pltpu.sync_copy(idx_hbm.at[pl.ds(cb, CH)], idx_v)
        pltpu.async_copy(w_hbm.at[idx_v], rows_v, sem).wait()
        pltpu.sync_copy(x_hbm.at[pl.ds(cb, CH)], x_v)

        def tok(t, carry2):
            s = None
            for j in range(DIM // 16):
                sl = pl.ds(16 * j, 16)
                xv = x_v[t, sl]
                wv = rows_v[t, sl]
                dv = wv - xv
                rows_v[t, sl] = xv + dv
                sq = dv * dv
                s = sq if s is None else s + sq
            acc_v[...] = acc_v[...] + s
            return carry2

        lax.fori_loop(0, CH, tok, 0)
        pltpu.sync_copy(rows_v, qst_hbm.at[pl.ds(cb, CH)])
        return carry

    lax.fori_loop(0, BPW // CH, chunk, 0)
    pltpu.sync_copy(acc_v, loss_hbm.at[wid])


def kernel(inputs, W):
    encoding_indices = _argmin_call(inputs, W.T)
    quantized_st, loss_parts = _sc_phase2(
        W, inputs, encoding_indices.reshape(N_TOK))
    mse = jnp.sum(loss_parts) / (N_TOK * DIM)
    vq_loss = mse + COMMIT * mse
    return (quantized_st, vq_loss, encoding_indices)
